# baseline (speedup 1.0000x reference)
import jax
import jax.numpy as jnp
from jax import lax
from jax.experimental import pallas as pl
from jax.experimental.pallas import tpu as pltpu
from jax.experimental.pallas import tpu_sc as plsc

B, V, F_IN = 2, 4096, 64
K, S_DIM, F_LR, F_OUT = 16, 4, 12, 18
FP = 16
TI = 2048
NW = 32
VPW = V // NW

_HI = lax.Precision.HIGHEST


def _topk_kernel(xt_ref, xT_ref, ws_ref, bs_ref, wsT_ref, bsc_ref,
                 wf_ref, bf_ref, wo1_ref, bo_ref,
                 idx_ref, w_ref, flr_ref, acc_ref):
    i = pl.program_id(0)
    XT = xT_ref[...]
    ST = jnp.dot(wsT_ref[...], XT, precision=_HI) + bsc_ref[...]

    Xt = xt_ref[...]
    St = jnp.dot(Xt, ws_ref[...], precision=_HI) + bs_ref[...]

    Ft = jnp.dot(Xt, wf_ref[...], precision=_HI) + bf_ref[...]
    flr_ref[...] = jnp.concatenate(
        [Ft, jnp.zeros((TI, FP - F_LR), jnp.float32)], axis=1)

    d2 = jnp.zeros((TI, V), jnp.float32)
    for s in range(S_DIM):
        diff = St[:, s:s + 1] - ST[s:s + 1, :]
        d2 = d2 + diff * diff

    col = lax.broadcasted_iota(jnp.int32, (TI, V), 1)
    row = i * TI + lax.broadcasted_iota(jnp.int32, (TI, V), 0)
    work = jnp.where(col == row, jnp.inf, d2)

    kcol = lax.broadcasted_iota(jnp.int32, (TI, K), 1)
    idx_acc = jnp.zeros((TI, K), jnp.int32)
    w_acc = jnp.zeros((TI, K), jnp.float32)
    colf = col.astype(jnp.float32)
    fV = jnp.float32(V)
    for k in range(K):
        m = jnp.min(work, axis=1, keepdims=True)
        eq = work == m
        ivf = jnp.min(jnp.where(eq, colf, fV), axis=1, keepdims=True)
        iv = ivf.astype(jnp.int32)
        work = jnp.where(eq, jnp.inf, work)
        idx_acc = jnp.where(kcol == k, iv, idx_acc)
        w_acc = jnp.where(kcol == k, jnp.exp(-10.0 * m), w_acc)

    idx_ref[...] = idx_acc
    w_ref[...] = w_acc
    acc_ref[...] = jnp.dot(Xt, wo1_ref[...], precision=_HI) + bo_ref[...]


def _run_topk(x, xT, W_s, b_s, W_flr, b_flr, Wo1, b_out):
    grid = (V // TI,)
    tile = lambda i: (i, 0)
    cst = lambda i: (0, 0)
    return pl.pallas_call(
        _topk_kernel,
        grid=grid,
        in_specs=[
            pl.BlockSpec((TI, F_IN), tile),
            pl.BlockSpec((F_IN, V), cst),
            pl.BlockSpec((F_IN, S_DIM), cst),
            pl.BlockSpec((1, S_DIM), cst),
            pl.BlockSpec((S_DIM, F_IN), cst),
            pl.BlockSpec((S_DIM, 1), cst),
            pl.BlockSpec((F_IN, F_LR), cst),
            pl.BlockSpec((1, F_LR), cst),
            pl.BlockSpec((F_IN, F_OUT), cst),
            pl.BlockSpec((1, F_OUT), cst),
        ],
        out_specs=[
            pl.BlockSpec((TI, K), tile),
            pl.BlockSpec((TI, K), tile),
            pl.BlockSpec((TI, FP), tile),
            pl.BlockSpec((TI, F_OUT), tile),
        ],
        out_shape=[
            jax.ShapeDtypeStruct((V, K), jnp.int32),
            jax.ShapeDtypeStruct((V, K), jnp.float32),
            jax.ShapeDtypeStruct((V, FP), jnp.float32),
            jax.ShapeDtypeStruct((V, F_OUT), jnp.float32),
        ],
    )(x, xT, W_s, b_s[None, :], W_s.T, b_s[:, None],
      W_flr, b_flr[None, :], Wo1, b_out[None, :])


def _sc_agg_body(idx_h, w_h, flr_h, outm_h, outx_h,
                 idx_v, w_v, flr_v, rm_v, rx_v):
    nc = 2
    wid = lax.axis_index("s") * nc + lax.axis_index("c")
    base = wid * VPW
    pltpu.sync_copy(idx_h.at[pl.ds(base * K, VPW * K)], idx_v)
    pltpu.sync_copy(w_h.at[pl.ds(base * K, VPW * K)], w_v)
    pltpu.sync_copy(flr_h, flr_v)

    lane = lax.broadcasted_iota(jnp.int32, (16,), 0)

    def body(v, _):
        acc_m = jnp.zeros((16,), jnp.float32)
        acc_x = jnp.full((16,), -jnp.inf, jnp.float32)
        ivec = idx_v[pl.ds(v * K, K)]
        wvec = w_v[pl.ds(v * K, K)]
        for k in range(K):
            j = ivec[k]
            wk = wvec[k]
            vals = plsc.load_gather(flr_v, [j * FP + lane])
            t = wk * vals
            acc_m = acc_m + t
            acc_x = jnp.maximum(acc_x, t)
        rm_v[pl.ds(v * FP, FP)] = acc_m * (1.0 / K)
        rx_v[pl.ds(v * FP, FP)] = acc_x
        return 0

    lax.fori_loop(0, VPW, body, 0)
    pltpu.sync_copy(rm_v, outm_h.at[pl.ds(base * FP, VPW * FP)])
    pltpu.sync_copy(rx_v, outx_h.at[pl.ds(base * FP, VPW * FP)])


def _run_sc_agg(idx, w, flr):
    mesh = plsc.VectorSubcoreMesh(core_axis_name="c", subcore_axis_name="s")
    fn = pl.kernel(
        _sc_agg_body,
        mesh=mesh,
        compiler_params=pltpu.CompilerParams(needs_layout_passes=False),
        out_type=[
            jax.ShapeDtypeStruct((V * FP,), jnp.float32),
            jax.ShapeDtypeStruct((V * FP,), jnp.float32),
        ],
        scratch_types=[
            pltpu.VMEM((VPW * K,), jnp.int32),
            pltpu.VMEM((VPW * K,), jnp.float32),
            pltpu.VMEM((V * FP,), jnp.float32),
            pltpu.VMEM((VPW * FP,), jnp.float32),
            pltpu.VMEM((VPW * FP,), jnp.float32),
        ],
    )
    return fn(idx.reshape(V * K), w.reshape(V * K), flr.reshape(V * FP))


def _final_kernel(acc_ref, m_ref, x_ref, wo2_ref, wo3_ref, out_ref):
    acc = (acc_ref[...]
           + jnp.dot(m_ref[...], wo2_ref[...], precision=_HI)
           + jnp.dot(x_ref[...], wo3_ref[...], precision=_HI))
    out_ref[...] = jnp.tanh(acc)


def _run_final(acc, aggm, aggx, Wo2p, Wo3p):
    return pl.pallas_call(
        _final_kernel,
        out_shape=jax.ShapeDtypeStruct((V, F_OUT), jnp.float32),
    )(acc, aggm, aggx, Wo2p, Wo3p)


def kernel(inputs, W_s, b_s, W_flr, b_flr, W_out, b_out):
    Wo1 = W_out[:F_IN]
    Wo2p = jnp.concatenate(
        [W_out[F_IN:F_IN + F_LR], jnp.zeros((FP - F_LR, F_OUT), jnp.float32)])
    Wo3p = jnp.concatenate(
        [W_out[F_IN + F_LR:], jnp.zeros((FP - F_LR, F_OUT), jnp.float32)])
    inputs_T = jnp.swapaxes(inputs, 1, 2)

    outs = []
    for b in range(B):
        idx, w, flr, acc = _run_topk(inputs[b], inputs_T[b], W_s, b_s,
                                     W_flr, b_flr, Wo1, b_out)
        aggm, aggx = _run_sc_agg(idx, w, flr)
        outs.append(_run_final(acc, aggm.reshape(V, FP),
                               aggx.reshape(V, FP), Wo2p, Wo3p))
    return jnp.stack(outs, axis=0)


# speedup vs baseline: 1.0614x; 1.0614x over previous
import functools
import jax
import jax.numpy as jnp
from jax import lax
from jax.experimental import pallas as pl
from jax.experimental.pallas import tpu as pltpu
from jax.experimental.pallas import tpu_sc as plsc

B, V, F_IN = 2, 4096, 64
K, S_DIM, F_LR, F_OUT = 16, 4, 12, 18
FP = 16
TI = 2048
NW = 32
VPW = (B * V) // NW

_HI = lax.Precision.HIGHEST


def _topk_kernel(xt_ref, xT_ref, ws_ref, bs_ref, wsT_ref, bsc_ref,
                 wf_ref, bf_ref, wo1_ref, bo_ref,
                 idx_ref, w_ref, flr_ref, acc_ref):
    i = pl.program_id(1)
    XT = xT_ref[0]
    ST = jnp.dot(wsT_ref[...], XT, precision=_HI) + bsc_ref[...]

    Xt = xt_ref[0]
    St = jnp.dot(Xt, ws_ref[...], precision=_HI) + bs_ref[...]

    Ft = jnp.dot(Xt, wf_ref[...], precision=_HI) + bf_ref[...]
    flr_ref[0] = jnp.concatenate(
        [Ft, jnp.zeros((TI, FP - F_LR), jnp.float32)], axis=1)

    d2 = jnp.zeros((TI, V), jnp.float32)
    for s in range(S_DIM):
        diff = St[:, s:s + 1] - ST[s:s + 1, :]
        d2 = d2 + diff * diff

    col = lax.broadcasted_iota(jnp.int32, (TI, V), 1)
    row = i * TI + lax.broadcasted_iota(jnp.int32, (TI, V), 0)
    work = jnp.where(col == row, jnp.inf, d2)

    kcol = lax.broadcasted_iota(jnp.int32, (TI, K), 1)
    idx_acc = jnp.zeros((TI, K), jnp.int32)
    w_acc = jnp.zeros((TI, K), jnp.float32)
    colf = col.astype(jnp.float32)
    fV = jnp.float32(V)
    for k in range(K):
        m = jnp.min(work, axis=1, keepdims=True)
        eq = work == m
        ivf = jnp.min(jnp.where(eq, colf, fV), axis=1, keepdims=True)
        iv = ivf.astype(jnp.int32)
        work = jnp.where(eq, jnp.inf, work)
        idx_acc = jnp.where(kcol == k, iv, idx_acc)
        w_acc = jnp.where(kcol == k, jnp.exp(-10.0 * m), w_acc)

    idx_ref[0] = idx_acc
    w_ref[0] = w_acc
    acc_ref[0] = jnp.dot(Xt, wo1_ref[...], precision=_HI) + bo_ref[...]


def _run_topk(inputs, inputs_T, W_s, b_s, W_flr, b_flr, Wo1, b_out):
    grid = (B, V // TI)
    tile = lambda b, i: (b, i, 0)
    cst = lambda b, i: (0, 0)
    return pl.pallas_call(
        _topk_kernel,
        grid=grid,
        in_specs=[
            pl.BlockSpec((1, TI, F_IN), tile),
            pl.BlockSpec((1, F_IN, V), lambda b, i: (b, 0, 0)),
            pl.BlockSpec((F_IN, S_DIM), cst),
            pl.BlockSpec((1, S_DIM), cst),
            pl.BlockSpec((S_DIM, F_IN), cst),
            pl.BlockSpec((S_DIM, 1), cst),
            pl.BlockSpec((F_IN, F_LR), cst),
            pl.BlockSpec((1, F_LR), cst),
            pl.BlockSpec((F_IN, F_OUT), cst),
            pl.BlockSpec((1, F_OUT), cst),
        ],
        out_specs=[
            pl.BlockSpec((1, TI, K), tile),
            pl.BlockSpec((1, TI, K), tile),
            pl.BlockSpec((1, TI, FP), tile),
            pl.BlockSpec((1, TI, F_OUT), tile),
        ],
        out_shape=[
            jax.ShapeDtypeStruct((B, V, K), jnp.int32),
            jax.ShapeDtypeStruct((B, V, K), jnp.float32),
            jax.ShapeDtypeStruct((B, V, FP), jnp.float32),
            jax.ShapeDtypeStruct((B, V, F_OUT), jnp.float32),
        ],
    )(inputs, inputs_T, W_s, b_s[None, :], W_s.T, b_s[:, None],
      W_flr, b_flr[None, :], Wo1, b_out[None, :])


def _sc_agg_body(idx_h, w_h, flr_h, outm_h, outx_h,
                 idx_v, w_v, flr_v, rm_v, rx_v):
    nc = 2
    wid = lax.axis_index("s") * nc + lax.axis_index("c")
    base = wid * VPW
    b = base // V
    pltpu.sync_copy(idx_h.at[pl.ds(base * K, VPW * K)], idx_v)
    pltpu.sync_copy(w_h.at[pl.ds(base * K, VPW * K)], w_v)
    pltpu.sync_copy(flr_h.at[pl.ds(b * V * FP, V * FP)], flr_v)

    lane = lax.broadcasted_iota(jnp.int32, (16,), 0)

    def body(v, _):
        acc_m = jnp.zeros((16,), jnp.float32)
        acc_x = jnp.full((16,), -jnp.inf, jnp.float32)
        ivec = idx_v[pl.ds(v * K, K)]
        wvec = w_v[pl.ds(v * K, K)]
        for k in range(K):
            j = ivec[k]
            wk = wvec[k]
            vals = plsc.load_gather(flr_v, [j * FP + lane])
            t = wk * vals
            acc_m = acc_m + t
            acc_x = jnp.maximum(acc_x, t)
        rm_v[pl.ds(v * FP, FP)] = acc_m * (1.0 / K)
        rx_v[pl.ds(v * FP, FP)] = acc_x
        return 0

    lax.fori_loop(0, VPW, body, 0)
    pltpu.sync_copy(rm_v, outm_h.at[pl.ds(base * FP, VPW * FP)])
    pltpu.sync_copy(rx_v, outx_h.at[pl.ds(base * FP, VPW * FP)])


def _run_sc_agg(idx, w, flr):
    mesh = plsc.VectorSubcoreMesh(core_axis_name="c", subcore_axis_name="s")
    fn = pl.kernel(
        _sc_agg_body,
        mesh=mesh,
        compiler_params=pltpu.CompilerParams(needs_layout_passes=False),
        out_type=[
            jax.ShapeDtypeStruct((B * V * FP,), jnp.float32),
            jax.ShapeDtypeStruct((B * V * FP,), jnp.float32),
        ],
        scratch_types=[
            pltpu.VMEM((VPW * K,), jnp.int32),
            pltpu.VMEM((VPW * K,), jnp.float32),
            pltpu.VMEM((V * FP,), jnp.float32),
            pltpu.VMEM((VPW * FP,), jnp.float32),
            pltpu.VMEM((VPW * FP,), jnp.float32),
        ],
    )
    return fn(idx.reshape(B * V * K), w.reshape(B * V * K),
              flr.reshape(B * V * FP))


def _final_kernel(acc_ref, m_ref, x_ref, wo2_ref, wo3_ref, out_ref):
    acc = (acc_ref[0]
           + jnp.dot(m_ref[0], wo2_ref[...], precision=_HI)
           + jnp.dot(x_ref[0], wo3_ref[...], precision=_HI))
    out_ref[0] = jnp.tanh(acc)


def _run_final(acc, aggm, aggx, Wo2p, Wo3p):
    cst = lambda b: (0, 0)
    return pl.pallas_call(
        _final_kernel,
        grid=(B,),
        in_specs=[
            pl.BlockSpec((1, V, F_OUT), lambda b: (b, 0, 0)),
            pl.BlockSpec((1, V, FP), lambda b: (b, 0, 0)),
            pl.BlockSpec((1, V, FP), lambda b: (b, 0, 0)),
            pl.BlockSpec((FP, F_OUT), cst),
            pl.BlockSpec((FP, F_OUT), cst),
        ],
        out_specs=pl.BlockSpec((1, V, F_OUT), lambda b: (b, 0, 0)),
        out_shape=jax.ShapeDtypeStruct((B, V, F_OUT), jnp.float32),
    )(acc, aggm, aggx, Wo2p, Wo3p)


def kernel(inputs, W_s, b_s, W_flr, b_flr, W_out, b_out):
    Wo1 = W_out[:F_IN]
    Wo2p = jnp.concatenate(
        [W_out[F_IN:F_IN + F_LR], jnp.zeros((FP - F_LR, F_OUT), jnp.float32)])
    Wo3p = jnp.concatenate(
        [W_out[F_IN + F_LR:], jnp.zeros((FP - F_LR, F_OUT), jnp.float32)])
    inputs_T = jnp.swapaxes(inputs, 1, 2)

    idx, w, flr, acc = _run_topk(inputs, inputs_T, W_s, b_s, W_flr, b_flr,
                                 Wo1, b_out)
    aggm, aggx = _run_sc_agg(idx, w, flr)
    return _run_final(acc, aggm.reshape(B, V, FP), aggx.reshape(B, V, FP),
                      Wo2p, Wo3p)


# speedup vs baseline: 1.0619x; 1.0005x over previous
import functools
import jax
import jax.numpy as jnp
from jax import lax
from jax.experimental import pallas as pl
from jax.experimental.pallas import tpu as pltpu
from jax.experimental.pallas import tpu_sc as plsc

B, V, F_IN = 2, 4096, 64
K, S_DIM, F_LR, F_OUT = 16, 4, 12, 18
FP = 16
TI = 2048
NW = 32
VPW = (B * V) // NW

_HI = lax.Precision.HIGHEST


def _topk_kernel(xt_ref, xT_ref, ws_ref, bs_ref, wsT_ref, bsc_ref,
                 wf_ref, bf_ref, wo1_ref, bo_ref,
                 idx_ref, w_ref, flr_ref, acc_ref):
    i = pl.program_id(1)
    XT = xT_ref[0]
    ST = jnp.dot(wsT_ref[...], XT, precision=_HI) + bsc_ref[...]

    Xt = xt_ref[0]
    St = jnp.dot(Xt, ws_ref[...], precision=_HI) + bs_ref[...]

    Ft = jnp.dot(Xt, wf_ref[...], precision=_HI) + bf_ref[...]
    flr_ref[0] = jnp.concatenate(
        [Ft, jnp.zeros((TI, FP - F_LR), jnp.float32)], axis=1)

    d2 = jnp.zeros((TI, V), jnp.float32)
    for s in range(S_DIM):
        diff = St[:, s:s + 1] - ST[s:s + 1, :]
        d2 = d2 + diff * diff

    col = lax.broadcasted_iota(jnp.int32, (TI, V), 1)
    row = i * TI + lax.broadcasted_iota(jnp.int32, (TI, V), 0)
    work = jnp.where(col == row, jnp.inf, d2)

    kcol = lax.broadcasted_iota(jnp.int32, (TI, K), 1)
    idx_acc = jnp.zeros((TI, K), jnp.int32)
    w_acc = jnp.zeros((TI, K), jnp.float32)
    colf = col.astype(jnp.float32)
    fV = jnp.float32(V)
    for k in range(K):
        m = jnp.min(work, axis=1, keepdims=True)
        eq = work == m
        ivf = jnp.min(jnp.where(eq, colf, fV), axis=1, keepdims=True)
        iv = ivf.astype(jnp.int32)
        if k < K - 1:
            work = jnp.where(eq, jnp.inf, work)
        idx_acc = jnp.where(kcol == k, iv, idx_acc)
        w_acc = jnp.where(kcol == k, jnp.exp(-10.0 * m), w_acc)

    idx_ref[0] = idx_acc
    w_ref[0] = w_acc
    acc_ref[0] = jnp.dot(Xt, wo1_ref[...], precision=_HI) + bo_ref[...]


def _run_topk(inputs, inputs_T, W_s, b_s, W_flr, b_flr, Wo1, b_out):
    grid = (B, V // TI)
    tile = lambda b, i: (b, i, 0)
    cst = lambda b, i: (0, 0)
    return pl.pallas_call(
        _topk_kernel,
        grid=grid,
        in_specs=[
            pl.BlockSpec((1, TI, F_IN), tile),
            pl.BlockSpec((1, F_IN, V), lambda b, i: (b, 0, 0)),
            pl.BlockSpec((F_IN, S_DIM), cst),
            pl.BlockSpec((1, S_DIM), cst),
            pl.BlockSpec((S_DIM, F_IN), cst),
            pl.BlockSpec((S_DIM, 1), cst),
            pl.BlockSpec((F_IN, F_LR), cst),
            pl.BlockSpec((1, F_LR), cst),
            pl.BlockSpec((F_IN, F_OUT), cst),
            pl.BlockSpec((1, F_OUT), cst),
        ],
        out_specs=[
            pl.BlockSpec((1, TI, K), tile),
            pl.BlockSpec((1, TI, K), tile),
            pl.BlockSpec((1, TI, FP), tile),
            pl.BlockSpec((1, TI, F_OUT), tile),
        ],
        out_shape=[
            jax.ShapeDtypeStruct((B, V, K), jnp.int32),
            jax.ShapeDtypeStruct((B, V, K), jnp.float32),
            jax.ShapeDtypeStruct((B, V, FP), jnp.float32),
            jax.ShapeDtypeStruct((B, V, F_OUT), jnp.float32),
        ],
    )(inputs, inputs_T, W_s, b_s[None, :], W_s.T, b_s[:, None],
      W_flr, b_flr[None, :], Wo1, b_out[None, :])


def _sc_agg_body(idx_h, w_h, flr_h, outm_h, outx_h,
                 idx_v, w_v, flr_v, rm_v, rx_v):
    nc = 2
    wid = lax.axis_index("s") * nc + lax.axis_index("c")
    base = wid * VPW
    b = base // V
    pltpu.sync_copy(idx_h.at[pl.ds(base * K, VPW * K)], idx_v)
    pltpu.sync_copy(w_h.at[pl.ds(base * K, VPW * K)], w_v)
    pltpu.sync_copy(flr_h.at[pl.ds(b * V * FP, V * FP)], flr_v)

    lane = lax.broadcasted_iota(jnp.int32, (16,), 0)

    def body(v, _):
        acc_m = jnp.zeros((16,), jnp.float32)
        acc_x = jnp.full((16,), -jnp.inf, jnp.float32)
        ivec = idx_v[pl.ds(v * K, K)]
        wvec = w_v[pl.ds(v * K, K)]
        for k in range(K):
            j = ivec[k]
            wk = wvec[k]
            vals = plsc.load_gather(flr_v, [j * FP + lane])
            t = wk * vals
            acc_m = acc_m + t
            acc_x = jnp.maximum(acc_x, t)
        rm_v[pl.ds(v * FP, FP)] = acc_m * (1.0 / K)
        rx_v[pl.ds(v * FP, FP)] = acc_x
        return 0

    lax.fori_loop(0, VPW, body, 0)
    pltpu.sync_copy(rm_v, outm_h.at[pl.ds(base * FP, VPW * FP)])
    pltpu.sync_copy(rx_v, outx_h.at[pl.ds(base * FP, VPW * FP)])


def _run_sc_agg(idx, w, flr):
    mesh = plsc.VectorSubcoreMesh(core_axis_name="c", subcore_axis_name="s")
    fn = pl.kernel(
        _sc_agg_body,
        mesh=mesh,
        compiler_params=pltpu.CompilerParams(needs_layout_passes=False),
        out_type=[
            jax.ShapeDtypeStruct((B * V * FP,), jnp.float32),
            jax.ShapeDtypeStruct((B * V * FP,), jnp.float32),
        ],
        scratch_types=[
            pltpu.VMEM((VPW * K,), jnp.int32),
            pltpu.VMEM((VPW * K,), jnp.float32),
            pltpu.VMEM((V * FP,), jnp.float32),
            pltpu.VMEM((VPW * FP,), jnp.float32),
            pltpu.VMEM((VPW * FP,), jnp.float32),
        ],
    )
    return fn(idx.reshape(B * V * K), w.reshape(B * V * K),
              flr.reshape(B * V * FP))


def _final_kernel(acc_ref, m_ref, x_ref, wo2_ref, wo3_ref, out_ref):
    acc = (acc_ref[0]
           + jnp.dot(m_ref[0], wo2_ref[...], precision=_HI)
           + jnp.dot(x_ref[0], wo3_ref[...], precision=_HI))
    out_ref[0] = jnp.tanh(acc)


def _run_final(acc, aggm, aggx, Wo2p, Wo3p):
    cst = lambda b: (0, 0)
    return pl.pallas_call(
        _final_kernel,
        grid=(B,),
        in_specs=[
            pl.BlockSpec((1, V, F_OUT), lambda b: (b, 0, 0)),
            pl.BlockSpec((1, V, FP), lambda b: (b, 0, 0)),
            pl.BlockSpec((1, V, FP), lambda b: (b, 0, 0)),
            pl.BlockSpec((FP, F_OUT), cst),
            pl.BlockSpec((FP, F_OUT), cst),
        ],
        out_specs=pl.BlockSpec((1, V, F_OUT), lambda b: (b, 0, 0)),
        out_shape=jax.ShapeDtypeStruct((B, V, F_OUT), jnp.float32),
    )(acc, aggm, aggx, Wo2p, Wo3p)


def kernel(inputs, W_s, b_s, W_flr, b_flr, W_out, b_out):
    Wo1 = W_out[:F_IN]
    Wo2p = jnp.concatenate(
        [W_out[F_IN:F_IN + F_LR], jnp.zeros((FP - F_LR, F_OUT), jnp.float32)])
    Wo3p = jnp.concatenate(
        [W_out[F_IN + F_LR:], jnp.zeros((FP - F_LR, F_OUT), jnp.float32)])
    inputs_T = jnp.swapaxes(inputs, 1, 2)

    idx, w, flr, acc = _run_topk(inputs, inputs_T, W_s, b_s, W_flr, b_flr,
                                 Wo1, b_out)
    aggm, aggx = _run_sc_agg(idx, w, flr)
    return _run_final(acc, aggm.reshape(B, V, FP), aggx.reshape(B, V, FP),
                      Wo2p, Wo3p)
